# Initial kernel scaffold; baseline (speedup 1.0000x reference)
import jax
import jax.numpy as jnp
from jax.experimental import pallas as pl


def kernel(x):
    raise NotImplementedError("write your pallas kernel here")


# speedup vs baseline: 3.4701x; 3.4701x over previous
import functools

import jax
import jax.numpy as jnp
from jax.experimental import pallas as pl


def _knn_loss_block(x_row_ref, x_all_ref, out_ref, *, k, rows_per_blk):
    i = pl.program_id(0)
    xr = x_row_ref[:]
    xa = x_all_ref[:]
    n = xa.shape[0]

    sq_r = jnp.sum(xr * xr, axis=1)
    sq_a = jnp.sum(xa * xa, axis=1)
    prod = jax.lax.dot_general(
        xr, xa, (((1,), (1,)), ((), ())), preferred_element_type=jnp.float32
    )
    d2 = jnp.maximum(sq_r[:, None] + sq_a[None, :] - 2.0 * prod, 0.0)
    dist = jnp.sqrt(d2)

    row_ids = i * rows_per_blk + jax.lax.broadcasted_iota(
        jnp.int32, (rows_per_blk, n), 0
    )
    col_ids = jax.lax.broadcasted_iota(jnp.int32, (rows_per_blk, n), 1)
    diag = row_ids == col_ids

    distm = jnp.where(diag, jnp.inf, dist)
    s = jnp.where(diag, 0.0, jnp.exp(-dist))
    denom = jnp.sum(s, axis=1)

    m1 = jnp.min(distm, axis=1)
    eq = distm == m1[:, None]
    cnt = jnp.sum(eq.astype(jnp.float32), axis=1)
    m2_excl = jnp.min(jnp.where(eq, jnp.inf, distm), axis=1)
    m2 = jnp.where(cnt >= 2.0, m1, m2_excl)

    loss_rows = (m1 + m2) * (1.0 / k) + jnp.log(denom)
    part = jnp.sum(loss_rows)[None, None]

    @pl.when(i == 0)
    def _init():
        out_ref[:, :] = jnp.zeros((1, 1), jnp.float32)

    out_ref[:, :] += part


def kernel(x):
    n, d = x.shape
    rows_per_blk = 512
    out = pl.pallas_call(
        functools.partial(_knn_loss_block, k=2, rows_per_blk=rows_per_blk),
        grid=(n // rows_per_blk,),
        in_specs=[
            pl.BlockSpec((rows_per_blk, d), lambda i: (i, 0)),
            pl.BlockSpec((n, d), lambda i: (0, 0)),
        ],
        out_specs=pl.BlockSpec((1, 1), lambda i: (0, 0)),
        out_shape=jax.ShapeDtypeStruct((1, 1), jnp.float32),
    )(x, x)
    return out[0, 0] / n


# speedup vs baseline: 3.5555x; 1.0246x over previous
import functools

import jax
import jax.numpy as jnp
from jax.experimental import pallas as pl
from jax.experimental.pallas import tpu as pltpu

_BIG = 1e9


def _knn_loss_block(x_row_ref, xt_ref, pen_ref, out_ref, tm_ref, sqa_ref, *,
                    k, rows_per_blk):
    i = pl.program_id(0)

    @pl.when(i == 0)
    def _init_sqa():
        xt = xt_ref[:]
        sqa_ref[:, :] = jnp.sum(xt * xt, axis=0, keepdims=True)

    xr = x_row_ref[:]
    sq_r = jnp.sum(xr * xr, axis=1)
    p2 = jax.lax.dot_general(
        xr * -2.0, xt_ref[:], (((1,), (0,)), ((), ())),
        preferred_element_type=jnp.float32,
    )
    tm_ref[:, :] = p2 + sqa_ref[:, :]
    tm_ref[:, pl.ds(i * rows_per_blk, rows_per_blk)] += pen_ref[:]
    tm = tm_ref[:, :]

    m1q = jnp.min(tm, axis=1)
    eq = tm == m1q[:, None]
    cnt = jnp.sum(jnp.where(eq, 1.0, 0.0), axis=1)
    m2q = jnp.min(jnp.where(eq, _BIG, tm), axis=1)
    m2q = jnp.where(cnt >= 2.0, m1q, m2q)

    u = jnp.maximum(tm + sq_r[:, None], 0.0)
    s = jnp.exp(-jnp.sqrt(u))
    denom = jnp.sum(s, axis=1)

    d1 = jnp.sqrt(jnp.maximum(m1q + sq_r, 0.0))
    d2 = jnp.sqrt(jnp.maximum(m2q + sq_r, 0.0))
    loss_rows = (d1 + d2) * (1.0 / k) + jnp.log(denom)
    part = jnp.sum(loss_rows)[None, None]

    @pl.when(i == 0)
    def _init_out():
        out_ref[:, :] = jnp.zeros((1, 1), jnp.float32)

    out_ref[:, :] += part


def kernel(x):
    n, d = x.shape
    rows_per_blk = 512
    pen = _BIG * jnp.eye(rows_per_blk, dtype=jnp.float32)
    out = pl.pallas_call(
        functools.partial(_knn_loss_block, k=2, rows_per_blk=rows_per_blk),
        grid=(n // rows_per_blk,),
        in_specs=[
            pl.BlockSpec((rows_per_blk, d), lambda i: (i, 0)),
            pl.BlockSpec((d, n), lambda i: (0, 0)),
            pl.BlockSpec((rows_per_blk, rows_per_blk), lambda i: (0, 0)),
        ],
        out_specs=pl.BlockSpec((1, 1), lambda i: (0, 0)),
        out_shape=jax.ShapeDtypeStruct((1, 1), jnp.float32),
        scratch_shapes=[
            pltpu.VMEM((rows_per_blk, n), jnp.float32),
            pltpu.VMEM((1, n), jnp.float32),
        ],
    )(x, x.T, pen)
    return out[0, 0] / n
